# Initial kernel scaffold; baseline (speedup 1.0000x reference)
import functools

import jax
import jax.numpy as jnp
from jax.experimental import pallas as pl
from jax.experimental.pallas import tpu as pltpu

NUM_EXPERTS = 8
TOP_K = 2
D_MODEL = 1024
D_FF = 4096
N_TOKENS = 2048

FF_CHUNK = 1024
FF_STEPS = D_FF // FF_CHUNK


def _gate_body(x_ref, wg_ref, id1_ref, id2_ref, w1_ref, w2_ref):
    x = x_ref[...]
    wg = wg_ref[...]
    logits = jnp.dot(x, wg, preferred_element_type=jnp.float32,
                     precision=jax.lax.Precision.HIGHEST)
    iota = jax.lax.broadcasted_iota(jnp.int32, logits.shape, 1)
    m1 = jnp.max(logits, axis=1, keepdims=True)
    am1 = jnp.min(jnp.where(logits == m1, iota, NUM_EXPERTS), axis=1,
                  keepdims=True)
    masked = jnp.where(iota == am1, -jnp.inf, logits)
    m2 = jnp.max(masked, axis=1, keepdims=True)
    am2 = jnp.min(jnp.where(masked == m2, iota, NUM_EXPERTS), axis=1,
                  keepdims=True)
    z = jnp.exp(m2 - m1)
    id1_ref[...] = am1
    id2_ref[...] = am2
    w1_ref[...] = 1.0 / (1.0 + z)
    w2_ref[...] = z / (1.0 + z)


def _gate(inputs, W_gate):
    return pl.pallas_call(
        _gate_body,
        out_shape=(
            jax.ShapeDtypeStruct((N_TOKENS, 1), jnp.int32),
            jax.ShapeDtypeStruct((N_TOKENS, 1), jnp.int32),
            jax.ShapeDtypeStruct((N_TOKENS, 1), jnp.float32),
            jax.ShapeDtypeStruct((N_TOKENS, 1), jnp.float32),
        ),
    )(inputs, W_gate)


def _dense_body(id1_ref, id2_ref, w1_ref, w2_ref, x_ref, w1c_ref, w2c_ref,
                out_ref):
    e = pl.program_id(0)
    c = pl.program_id(1)

    @pl.when(jnp.logical_and(e == 0, c == 0))
    def _init():
        out_ref[...] = jnp.zeros_like(out_ref)

    xb = x_ref[...].astype(jnp.bfloat16)
    w1b = w1c_ref[0].astype(jnp.bfloat16)
    w2b = w2c_ref[0].astype(jnp.bfloat16)
    h = jnp.dot(xb, w1b, preferred_element_type=jnp.float32)
    hg = jax.nn.gelu(h).astype(jnp.bfloat16)
    y = jnp.dot(hg, w2b, preferred_element_type=jnp.float32)
    w_e = (w1_ref[...] * (id1_ref[...] == e).astype(jnp.float32)
           + w2_ref[...] * (id2_ref[...] == e).astype(jnp.float32))
    out_ref[...] += w_e * y


def _dense_experts(id1, id2, w1, w2, inputs, W1, W2):
    grid = (NUM_EXPERTS, FF_STEPS)
    return pl.pallas_call(
        _dense_body,
        grid=grid,
        in_specs=[
            pl.BlockSpec((N_TOKENS, 1), lambda e, c: (0, 0)),
            pl.BlockSpec((N_TOKENS, 1), lambda e, c: (0, 0)),
            pl.BlockSpec((N_TOKENS, 1), lambda e, c: (0, 0)),
            pl.BlockSpec((N_TOKENS, 1), lambda e, c: (0, 0)),
            pl.BlockSpec((N_TOKENS, D_MODEL), lambda e, c: (0, 0)),
            pl.BlockSpec((1, D_MODEL, FF_CHUNK), lambda e, c: (e, 0, c)),
            pl.BlockSpec((1, FF_CHUNK, D_MODEL), lambda e, c: (e, c, 0)),
        ],
        out_specs=pl.BlockSpec((N_TOKENS, D_MODEL), lambda e, c: (0, 0)),
        out_shape=jax.ShapeDtypeStruct((N_TOKENS, D_MODEL), jnp.float32),
    )(id1, id2, w1, w2, inputs, W1, W2)


def kernel(inputs, W_gate, W1, W2):
    id1, id2, w1, w2 = _gate(inputs, W_gate)
    return _dense_experts(id1, id2, w1, w2, inputs, W1, W2)


# speedup vs baseline: 1.2903x; 1.2903x over previous
import functools

import jax
import jax.numpy as jnp
from jax.experimental import pallas as pl
from jax.experimental.pallas import tpu as pltpu

NUM_EXPERTS = 8
TOP_K = 2
D_MODEL = 1024
D_FF = 4096
N_TOKENS = 2048

FF_CHUNK = 1024
FF_STEPS = D_FF // FF_CHUNK


def _gate_body(x_ref, wg_ref, id1_ref, id2_ref, w1_ref, w2_ref):
    x = x_ref[...]
    wg = wg_ref[...]
    logits = jnp.dot(x, wg, preferred_element_type=jnp.float32)
    iota = jax.lax.broadcasted_iota(jnp.int32, logits.shape, 1)
    m1 = jnp.max(logits, axis=1, keepdims=True)
    am1 = jnp.min(jnp.where(logits == m1, iota, NUM_EXPERTS), axis=1,
                  keepdims=True)
    masked = jnp.where(iota == am1, -jnp.inf, logits)
    m2 = jnp.max(masked, axis=1, keepdims=True)
    am2 = jnp.min(jnp.where(masked == m2, iota, NUM_EXPERTS), axis=1,
                  keepdims=True)
    z = jnp.exp(m2 - m1)
    id1_ref[...] = am1
    id2_ref[...] = am2
    w1_ref[...] = 1.0 / (1.0 + z)
    w2_ref[...] = z / (1.0 + z)


def _gate(inputs, W_gate):
    return pl.pallas_call(
        _gate_body,
        out_shape=(
            jax.ShapeDtypeStruct((N_TOKENS, 1), jnp.int32),
            jax.ShapeDtypeStruct((N_TOKENS, 1), jnp.int32),
            jax.ShapeDtypeStruct((N_TOKENS, 1), jnp.float32),
            jax.ShapeDtypeStruct((N_TOKENS, 1), jnp.float32),
        ),
    )(inputs, W_gate)


def _dense_body(id1_ref, id2_ref, w1_ref, w2_ref, x_ref, w1c_ref, w2c_ref,
                out_ref):
    e = pl.program_id(0)
    c = pl.program_id(1)

    @pl.when(jnp.logical_and(e == 0, c == 0))
    def _init():
        out_ref[...] = jnp.zeros_like(out_ref)

    xb = x_ref[...].astype(jnp.bfloat16)
    w1b = w1c_ref[0].astype(jnp.bfloat16)
    w2b = w2c_ref[0].astype(jnp.bfloat16)
    h = jnp.dot(xb, w1b, preferred_element_type=jnp.float32)
    hg = jax.nn.gelu(h).astype(jnp.bfloat16)
    y = jnp.dot(hg, w2b, preferred_element_type=jnp.float32)
    w_e = (w1_ref[...] * (id1_ref[...] == e).astype(jnp.float32)
           + w2_ref[...] * (id2_ref[...] == e).astype(jnp.float32))
    out_ref[...] += w_e * y


def _dense_experts(id1, id2, w1, w2, inputs, W1, W2):
    grid = (NUM_EXPERTS, FF_STEPS)
    return pl.pallas_call(
        _dense_body,
        grid=grid,
        in_specs=[
            pl.BlockSpec((N_TOKENS, 1), lambda e, c: (0, 0)),
            pl.BlockSpec((N_TOKENS, 1), lambda e, c: (0, 0)),
            pl.BlockSpec((N_TOKENS, 1), lambda e, c: (0, 0)),
            pl.BlockSpec((N_TOKENS, 1), lambda e, c: (0, 0)),
            pl.BlockSpec((N_TOKENS, D_MODEL), lambda e, c: (0, 0)),
            pl.BlockSpec((1, D_MODEL, FF_CHUNK), lambda e, c: (e, 0, c)),
            pl.BlockSpec((1, FF_CHUNK, D_MODEL), lambda e, c: (e, c, 0)),
        ],
        out_specs=pl.BlockSpec((N_TOKENS, D_MODEL), lambda e, c: (0, 0)),
        out_shape=jax.ShapeDtypeStruct((N_TOKENS, D_MODEL), jnp.float32),
    )(id1, id2, w1, w2, inputs, W1, W2)


def kernel(inputs, W_gate, W1, W2):
    id1, id2, w1, w2 = _gate(inputs, W_gate)
    return _dense_experts(id1, id2, w1, w2, inputs, W1, W2)
